# Initial kernel scaffold; baseline (speedup 1.0000x reference)
import functools

import jax
import jax.numpy as jnp
from jax import lax
from jax.experimental import pallas as pl
from jax.experimental.pallas import tpu as pltpu, tpu_sc as plsc

N_NODES = 10000
N_EDGES = 320000
D_IN = 128
D_OUT = 128

DH = D_IN // 2
NS = 16
ROWS_PT = N_NODES // NS
EB = 80
EROWS = N_EDGES // EB
EROWS_PT = EROWS // NS


def _sc_aggregate(feat_s, src2, dst2):
    mesh = plsc.VectorSubcoreMesh(core_axis_name="c", subcore_axis_name="s")

    @functools.partial(
        pl.kernel,
        out_type=jax.ShapeDtypeStruct((2, N_NODES, DH), jnp.float32),
        mesh=mesh,
        scratch_types=[
            pltpu.VMEM_SHARED((N_NODES, DH), jnp.float32),
            pltpu.VMEM_SHARED((N_NODES, DH), jnp.float32),
            pltpu.VMEM_SHARED((N_NODES, 16), jnp.float32),
            pltpu.VMEM((EROWS_PT, EB), jnp.int32),
            pltpu.VMEM((EROWS_PT, EB), jnp.int32),
            pltpu.VMEM((EB, DH), jnp.float32),
            pltpu.VMEM((ROWS_PT, DH), jnp.float32),
            pltpu.VMEM((ROWS_PT, 16), jnp.float32),
            pltpu.VMEM((EB, 16), jnp.float32),
            pltpu.SemaphoreType.DMA,
        ],
    )
    def k(feat_hbm, src_hbm, dst_hbm, agg_hbm,
          feat_sp, acc_sp, deg_sp, src_v, dst_v, rows_v, stage_v, degb_v,
          ones_v, sem):
        c = lax.axis_index("c")
        s = lax.axis_index("s")
        r0 = s * ROWS_PT
        e0 = s * EROWS_PT

        pltpu.sync_copy(feat_hbm.at[c, pl.ds(r0, ROWS_PT)],
                        feat_sp.at[pl.ds(r0, ROWS_PT)])

        zf = jnp.zeros((16,), jnp.float32)

        def zero_stage(i, _):
            for j in range(DH // 16):
                stage_v[i, pl.ds(16 * j, 16)] = zf
            degb_v[i, pl.ds(0, 16)] = zf
            return 0

        lax.fori_loop(0, ROWS_PT, zero_stage, 0)
        pltpu.sync_copy(stage_v, acc_sp.at[pl.ds(r0, ROWS_PT)])
        pltpu.sync_copy(degb_v, deg_sp.at[pl.ds(r0, ROWS_PT)])

        of = jnp.ones((16,), jnp.float32)

        def fill_ones(i, _):
            ones_v[i, pl.ds(0, 16)] = of
            return 0

        lax.fori_loop(0, EB, fill_ones, 0)

        pltpu.sync_copy(src_hbm.at[pl.ds(e0, EROWS_PT)], src_v)
        pltpu.sync_copy(dst_hbm.at[pl.ds(e0, EROWS_PT)], dst_v)

        plsc.subcore_barrier()

        def edge_body(j, _):
            pltpu.async_copy(feat_sp.at[src_v.at[j]], rows_v, sem).wait()
            pltpu.sync_copy(rows_v, acc_sp.at[dst_v.at[j]], add=True)
            pltpu.sync_copy(ones_v, deg_sp.at[dst_v.at[j]], add=True)
            return 0

        lax.fori_loop(0, EROWS_PT, edge_body, 0)

        plsc.subcore_barrier()

        pltpu.sync_copy(acc_sp.at[pl.ds(r0, ROWS_PT)], stage_v)
        pltpu.sync_copy(deg_sp.at[pl.ds(r0, ROWS_PT)], degb_v)

        def div_body(i, _):
            dinv = 1.0 / jnp.maximum(degb_v[i], 1.0)
            for j in range(DH // 16):
                stage_v[i, pl.ds(16 * j, 16)] = stage_v[i, pl.ds(16 * j, 16)] * dinv
            return 0

        lax.fori_loop(0, ROWS_PT, div_body, 0)
        pltpu.sync_copy(stage_v, agg_hbm.at[c, pl.ds(r0, ROWS_PT)])

    return k(feat_s, src2, dst2)


def _tc_body(feat_ref, agg_ref, w1_ref, w2_ref, b_ref, o_ref):
    o_ref[...] = (
        jnp.dot(feat_ref[...], w1_ref[...], preferred_element_type=jnp.float32)
        + jnp.dot(agg_ref[...], w2_ref[...], preferred_element_type=jnp.float32)
        + b_ref[...]
    )


def _tc_linear(feat, agg, w1t, w2t, b2d):
    rb = 1000
    grid = (N_NODES // rb,)
    return pl.pallas_call(
        _tc_body,
        grid=grid,
        in_specs=[
            pl.BlockSpec((rb, D_IN), lambda i: (i, 0)),
            pl.BlockSpec((rb, D_IN), lambda i: (i, 0)),
            pl.BlockSpec((D_IN, D_OUT), lambda i: (0, 0)),
            pl.BlockSpec((D_IN, D_OUT), lambda i: (0, 0)),
            pl.BlockSpec((1, D_OUT), lambda i: (0, 0)),
        ],
        out_specs=pl.BlockSpec((rb, D_OUT), lambda i: (i, 0)),
        out_shape=jax.ShapeDtypeStruct((N_NODES, D_OUT), jnp.float32),
    )(feat, agg, w1t, w2t, b2d)


def kernel(feat, edge_index, W, b):
    src2 = edge_index[0].astype(jnp.int32).reshape(EROWS, EB)
    dst2 = edge_index[1].astype(jnp.int32).reshape(EROWS, EB)
    feat_s = jnp.stack([feat[:, :DH], feat[:, DH:]])
    agg = _sc_aggregate(feat_s, src2, dst2)
    agg_full = jnp.concatenate([agg[0], agg[1]], axis=1)
    w1t = W[:, :D_IN].T
    w2t = W[:, D_IN:].T
    return _tc_linear(feat, agg_full, w1t, w2t, b.reshape(1, D_OUT))


# speedup vs baseline: 6.9665x; 6.9665x over previous
import functools

import jax
import jax.numpy as jnp
from jax import lax
from jax.experimental import pallas as pl
from jax.experimental.pallas import tpu as pltpu, tpu_sc as plsc

N_NODES = 10000
N_EDGES = 320000
D_IN = 128
D_OUT = 128

DH = D_IN // 2
NS = 16
N_PAD = 10240
ROWS_PT = N_PAD // NS
EB = 80
EROWS = N_EDGES // EB
EROWS_PT = EROWS // NS
CH = 10
NCH = EROWS_PT // CH
RCH = 80
NRCH = ROWS_PT // RCH


def _sc_aggregate(feat_s, src3, dst3):
    mesh = plsc.VectorSubcoreMesh(core_axis_name="c", subcore_axis_name="s")

    @functools.partial(
        pl.kernel,
        out_type=jax.ShapeDtypeStruct((2, N_PAD, DH), jnp.float32),
        mesh=mesh,
        scratch_types=[
            pltpu.VMEM_SHARED((N_PAD, DH), jnp.float32),
            pltpu.VMEM_SHARED((N_PAD, DH), jnp.float32),
            pltpu.VMEM_SHARED((N_PAD, 16), jnp.float32),
            pltpu.VMEM((CH, EB), jnp.int32),
            pltpu.VMEM((CH, EB), jnp.int32),
            pltpu.VMEM((EB, DH), jnp.float32),
            pltpu.VMEM((RCH, DH), jnp.float32),
            pltpu.VMEM((RCH, 16), jnp.float32),
            pltpu.VMEM((EB, 16), jnp.float32),
            pltpu.SemaphoreType.DMA,
        ],
        compiler_params=pltpu.CompilerParams(use_tc_tiling_on_sc=False),
    )
    def k(feat_hbm, src_hbm, dst_hbm, agg_hbm,
          feat_sp, acc_sp, deg_sp, src_v, dst_v, rows_v, stage_v, degb_v,
          ones_v, sem):
        c = lax.axis_index("c")
        s = lax.axis_index("s")
        r0 = s * ROWS_PT

        pltpu.sync_copy(feat_hbm.at[c, pl.ds(r0, ROWS_PT)],
                        feat_sp.at[pl.ds(r0, ROWS_PT)])

        zf = jnp.zeros((16,), jnp.float32)

        def zero_stage(i, _):
            for j in range(DH // 16):
                stage_v[i, pl.ds(16 * j, 16)] = zf
            degb_v[i, pl.ds(0, 16)] = zf
            return 0

        lax.fori_loop(0, RCH, zero_stage, 0)

        def zero_copy(kk, _):
            pltpu.sync_copy(stage_v, acc_sp.at[pl.ds(r0 + kk * RCH, RCH)])
            pltpu.sync_copy(degb_v, deg_sp.at[pl.ds(r0 + kk * RCH, RCH)])
            return 0

        lax.fori_loop(0, NRCH, zero_copy, 0)

        of = jnp.ones((16,), jnp.float32)

        def fill_ones(i, _):
            ones_v[i, pl.ds(0, 16)] = of
            return 0

        lax.fori_loop(0, EB, fill_ones, 0)

        plsc.subcore_barrier()

        def chunk_body(g, _):
            pltpu.sync_copy(src_hbm.at[s, pl.ds(g * CH, CH)], src_v)
            pltpu.sync_copy(dst_hbm.at[s, pl.ds(g * CH, CH)], dst_v)

            def edge_body(j, _):
                pltpu.async_copy(feat_sp.at[src_v.at[j]], rows_v, sem).wait()
                pltpu.sync_copy(rows_v, acc_sp.at[dst_v.at[j]], add=True)
                pltpu.sync_copy(ones_v, deg_sp.at[dst_v.at[j]], add=True)
                return 0

            lax.fori_loop(0, CH, edge_body, 0)
            return 0

        lax.fori_loop(0, NCH, chunk_body, 0)

        plsc.subcore_barrier()

        def fin_chunk(kk, _):
            rr = r0 + kk * RCH
            pltpu.sync_copy(acc_sp.at[pl.ds(rr, RCH)], stage_v)
            pltpu.sync_copy(deg_sp.at[pl.ds(rr, RCH)], degb_v)

            def div_body(i, _):
                dinv = 1.0 / jnp.maximum(degb_v[i], 1.0)
                for j in range(DH // 16):
                    stage_v[i, pl.ds(16 * j, 16)] = (
                        stage_v[i, pl.ds(16 * j, 16)] * dinv)
                return 0

            lax.fori_loop(0, RCH, div_body, 0)
            pltpu.sync_copy(stage_v, agg_hbm.at[c, pl.ds(rr, RCH)])
            return 0

        lax.fori_loop(0, NRCH, fin_chunk, 0)

    return k(feat_s, src3, dst3)


def _tc_body(feat_ref, agg_ref, w1_ref, w2_ref, b_ref, o_ref):
    o_ref[...] = (
        jnp.dot(feat_ref[...], w1_ref[...], preferred_element_type=jnp.float32)
        + jnp.dot(agg_ref[...], w2_ref[...], preferred_element_type=jnp.float32)
        + b_ref[...]
    )


def _tc_linear(feat, agg, w1t, w2t, b2d):
    rb = 1000
    grid = (N_NODES // rb,)
    return pl.pallas_call(
        _tc_body,
        grid=grid,
        in_specs=[
            pl.BlockSpec((rb, D_IN), lambda i: (i, 0)),
            pl.BlockSpec((rb, D_IN), lambda i: (i, 0)),
            pl.BlockSpec((D_IN, D_OUT), lambda i: (0, 0)),
            pl.BlockSpec((D_IN, D_OUT), lambda i: (0, 0)),
            pl.BlockSpec((1, D_OUT), lambda i: (0, 0)),
        ],
        out_specs=pl.BlockSpec((rb, D_OUT), lambda i: (i, 0)),
        out_shape=jax.ShapeDtypeStruct((N_NODES, D_OUT), jnp.float32),
    )(feat, agg, w1t, w2t, b2d)


def kernel(feat, edge_index, W, b):
    src3 = edge_index[0].astype(jnp.int32).reshape(NS, EROWS_PT, EB)
    dst3 = edge_index[1].astype(jnp.int32).reshape(NS, EROWS_PT, EB)
    feat_s = jnp.pad(jnp.stack([feat[:, :DH], feat[:, DH:]]),
                     ((0, 0), (0, N_PAD - N_NODES), (0, 0)))
    agg = _sc_aggregate(feat_s, src3, dst3)
    agg_full = jnp.concatenate([agg[0, :N_NODES], agg[1, :N_NODES]], axis=1)
    w1t = W[:, :D_IN].T
    w2t = W[:, D_IN:].T
    return _tc_linear(feat, agg_full, w1t, w2t, b.reshape(1, D_OUT))


# speedup vs baseline: 8.9553x; 1.2855x over previous
import functools

import jax
import jax.numpy as jnp
from jax import lax
from jax.experimental import pallas as pl
from jax.experimental.pallas import tpu as pltpu, tpu_sc as plsc

N_NODES = 10000
N_EDGES = 320000
D_IN = 128
D_OUT = 128

DH = D_IN // 2
NS = 16
N_PAD = 10240
ROWS_PT = N_PAD // NS
EB = 80
EROWS = N_EDGES // EB
EROWS_PT = EROWS // NS
CH = 50
NCH = EROWS_PT // CH
RCH = 80
NRCH = ROWS_PT // RCH


def _sc_aggregate(feat_s, src3, dst3):
    mesh = plsc.VectorSubcoreMesh(core_axis_name="c", subcore_axis_name="s")

    @functools.partial(
        pl.kernel,
        out_type=jax.ShapeDtypeStruct((2, N_PAD, DH), jnp.float32),
        mesh=mesh,
        scratch_types=[
            pltpu.VMEM_SHARED((N_PAD, DH), jnp.float32),
            pltpu.VMEM_SHARED((N_PAD, DH), jnp.float32),
            pltpu.VMEM_SHARED((N_PAD, 16), jnp.float32),
            pltpu.VMEM((CH, EB), jnp.int32),
            pltpu.VMEM((CH, EB), jnp.int32),
            pltpu.VMEM((2, EB, DH), jnp.float32),
            pltpu.VMEM((RCH, DH), jnp.float32),
            pltpu.VMEM((RCH, 16), jnp.float32),
            pltpu.VMEM((EB, 16), jnp.float32),
            pltpu.SemaphoreType.DMA,
            pltpu.SemaphoreType.DMA,
            pltpu.SemaphoreType.DMA,
        ],
        compiler_params=pltpu.CompilerParams(use_tc_tiling_on_sc=False),
    )
    def k(feat_hbm, src_hbm, dst_hbm, agg_hbm,
          feat_sp, acc_sp, deg_sp, src_v, dst_v, rows2, stage_v, degb_v,
          ones_v, gsem, ssem, dsem):
        c = lax.axis_index("c")
        s = lax.axis_index("s")
        r0 = s * ROWS_PT

        pltpu.sync_copy(feat_hbm.at[c, pl.ds(r0, ROWS_PT)],
                        feat_sp.at[pl.ds(r0, ROWS_PT)])

        zf = jnp.zeros((16,), jnp.float32)

        def zero_stage(i, _):
            for j in range(DH // 16):
                stage_v[i, pl.ds(16 * j, 16)] = zf
            degb_v[i, pl.ds(0, 16)] = zf
            return 0

        lax.fori_loop(0, RCH, zero_stage, 0)

        def zero_copy(kk, _):
            pltpu.sync_copy(stage_v, acc_sp.at[pl.ds(r0 + kk * RCH, RCH)])
            pltpu.sync_copy(degb_v, deg_sp.at[pl.ds(r0 + kk * RCH, RCH)])
            return 0

        lax.fori_loop(0, NRCH, zero_copy, 0)

        of = jnp.ones((16,), jnp.float32)

        def fill_ones(i, _):
            ones_v[i, pl.ds(0, 16)] = of
            return 0

        lax.fori_loop(0, EB, fill_ones, 0)

        plsc.subcore_barrier()

        def chunk_body(g, _):
            pltpu.sync_copy(src_hbm.at[s, pl.ds(g * CH, CH)], src_v)
            pltpu.sync_copy(dst_hbm.at[s, pl.ds(g * CH, CH)], dst_v)

            pltpu.async_copy(feat_sp.at[src_v.at[0]], rows2.at[0], gsem)

            def edge_body(j, _):
                b = lax.rem(j, 2)
                nb = 1 - b
                pltpu.make_async_copy(
                    feat_sp.at[src_v.at[j]], rows2.at[b], gsem).wait()

                @pl.when(j >= 1)
                def _():
                    pltpu.make_async_copy(
                        rows2.at[nb], acc_sp.at[dst_v.at[j - 1]], ssem).wait()
                    pltpu.make_async_copy(
                        ones_v, deg_sp.at[dst_v.at[j - 1]], dsem).wait()

                @pl.when(j + 1 < CH)
                def _():
                    pltpu.async_copy(
                        feat_sp.at[src_v.at[j + 1]], rows2.at[nb], gsem)

                pltpu.async_copy(
                    rows2.at[b], acc_sp.at[dst_v.at[j]], ssem, add=True)
                pltpu.async_copy(
                    ones_v, deg_sp.at[dst_v.at[j]], dsem, add=True)
                return 0

            lax.fori_loop(0, CH, edge_body, 0)
            lb = (CH - 1) % 2
            pltpu.make_async_copy(
                rows2.at[lb], acc_sp.at[dst_v.at[CH - 1]], ssem).wait()
            pltpu.make_async_copy(
                ones_v, deg_sp.at[dst_v.at[CH - 1]], dsem).wait()
            return 0

        lax.fori_loop(0, NCH, chunk_body, 0)

        plsc.subcore_barrier()

        def fin_chunk(kk, _):
            rr = r0 + kk * RCH
            pltpu.sync_copy(acc_sp.at[pl.ds(rr, RCH)], stage_v)
            pltpu.sync_copy(deg_sp.at[pl.ds(rr, RCH)], degb_v)

            def div_body(i, _):
                dinv = 1.0 / jnp.maximum(degb_v[i], 1.0)
                for j in range(DH // 16):
                    stage_v[i, pl.ds(16 * j, 16)] = (
                        stage_v[i, pl.ds(16 * j, 16)] * dinv)
                return 0

            lax.fori_loop(0, RCH, div_body, 0)
            pltpu.sync_copy(stage_v, agg_hbm.at[c, pl.ds(rr, RCH)])
            return 0

        lax.fori_loop(0, NRCH, fin_chunk, 0)

    return k(feat_s, src3, dst3)


def _tc_body(feat_ref, agg_ref, w1_ref, w2_ref, b_ref, o_ref):
    o_ref[...] = (
        jnp.dot(feat_ref[...], w1_ref[...], preferred_element_type=jnp.float32)
        + jnp.dot(agg_ref[...], w2_ref[...], preferred_element_type=jnp.float32)
        + b_ref[...]
    )


def _tc_linear(feat, agg, w1t, w2t, b2d):
    rb = 1000
    grid = (N_NODES // rb,)
    return pl.pallas_call(
        _tc_body,
        grid=grid,
        in_specs=[
            pl.BlockSpec((rb, D_IN), lambda i: (i, 0)),
            pl.BlockSpec((rb, D_IN), lambda i: (i, 0)),
            pl.BlockSpec((D_IN, D_OUT), lambda i: (0, 0)),
            pl.BlockSpec((D_IN, D_OUT), lambda i: (0, 0)),
            pl.BlockSpec((1, D_OUT), lambda i: (0, 0)),
        ],
        out_specs=pl.BlockSpec((rb, D_OUT), lambda i: (i, 0)),
        out_shape=jax.ShapeDtypeStruct((N_NODES, D_OUT), jnp.float32),
    )(feat, agg, w1t, w2t, b2d)


def kernel(feat, edge_index, W, b):
    src3 = edge_index[0].astype(jnp.int32).reshape(NS, EROWS_PT, EB)
    dst3 = edge_index[1].astype(jnp.int32).reshape(NS, EROWS_PT, EB)
    feat_s = jnp.pad(jnp.stack([feat[:, :DH], feat[:, DH:]]),
                     ((0, 0), (0, N_PAD - N_NODES), (0, 0)))
    agg = _sc_aggregate(feat_s, src3, dst3)
    agg_full = jnp.concatenate([agg[0, :N_NODES], agg[1, :N_NODES]], axis=1)
    w1t = W[:, :D_IN].T
    w2t = W[:, D_IN:].T
    return _tc_linear(feat, agg_full, w1t, w2t, b.reshape(1, D_OUT))


# speedup vs baseline: 9.5788x; 1.0696x over previous
import functools

import jax
import jax.numpy as jnp
from jax import lax
from jax.experimental import pallas as pl
from jax.experimental.pallas import tpu as pltpu, tpu_sc as plsc

N_NODES = 10000
N_EDGES = 320000
D_IN = 128
D_OUT = 128

DH = D_IN // 2
NS = 16
N_PAD = 10240
ROWS_PT = N_PAD // NS
EB = 80
EROWS = N_EDGES // EB
EROWS_PT = EROWS // NS
CH = 50
NCH = EROWS_PT // CH
RCH = 80
NRCH = ROWS_PT // RCH
DROWS = N_PAD // 16
IROWS = N_PAD // 2048


def _sc_aggregate(feat_s, src3, dst3):
    mesh = plsc.VectorSubcoreMesh(core_axis_name="c", subcore_axis_name="s")

    @functools.partial(
        pl.kernel,
        out_type=(
            jax.ShapeDtypeStruct((2, N_PAD, DH), jnp.float32),
            jax.ShapeDtypeStruct((DROWS, 16), jnp.float32),
        ),
        mesh=mesh,
        scratch_types=[
            pltpu.VMEM_SHARED((N_PAD, DH), jnp.float32),
            pltpu.VMEM_SHARED((N_PAD, DH), jnp.float32),
            pltpu.VMEM_SHARED((DROWS, 16), jnp.float32),
            pltpu.VMEM((CH, EB), jnp.int32),
            pltpu.VMEM((CH, EB), jnp.int32),
            pltpu.VMEM((2, EB, DH), jnp.float32),
            pltpu.VMEM((RCH, DH), jnp.float32),
            pltpu.VMEM((N_PAD,), jnp.float32),
            pltpu.VMEM((DROWS, 16), jnp.float32),
            pltpu.VMEM((IROWS, 128), jnp.int32),
            pltpu.SemaphoreType.DMA,
            pltpu.SemaphoreType.DMA,
        ],
        compiler_params=pltpu.CompilerParams(
            use_tc_tiling_on_sc=False, needs_layout_passes=False),
    )
    def k(feat_hbm, src_hbm, dst_hbm, agg_hbm, deg_hbm,
          feat_sp, acc_sp, deg_sp, src_v, dst_v, rows2, zero_v, deg_l,
          deg_l2, idx128, gsem, ssem):
        c = lax.axis_index("c")
        s = lax.axis_index("s")
        r0 = s * ROWS_PT

        pltpu.sync_copy(feat_hbm.at[c, pl.ds(r0, ROWS_PT)],
                        feat_sp.at[pl.ds(r0, ROWS_PT)])

        zf = jnp.zeros((16,), jnp.float32)

        def zero_stage(i, _):
            for j in range(DH // 16):
                zero_v[i, pl.ds(16 * j, 16)] = zf
            return 0

        lax.fori_loop(0, RCH, zero_stage, 0)

        def zero_hist(i, _):
            deg_l[pl.ds(16 * i, 16)] = zf
            deg_l2[i] = zf
            return 0

        lax.fori_loop(0, DROWS, zero_hist, 0)

        iot = lax.iota(jnp.int32, 16)

        def fill_iota(q, _):
            for t in range(8):
                idx128[q, pl.ds(16 * t, 16)] = iot + q * 128 + 16 * t
            return 0

        lax.fori_loop(0, IROWS, fill_iota, 0)

        def zero_copy(kk, _):
            pltpu.sync_copy(zero_v, acc_sp.at[pl.ds(r0 + kk * RCH, RCH)])
            return 0

        lax.fori_loop(0, NRCH, zero_copy, 0)
        pltpu.sync_copy(deg_l2.at[pl.ds(s * (DROWS // NS), DROWS // NS)],
                        deg_sp.at[pl.ds(s * (DROWS // NS), DROWS // NS)])

        plsc.subcore_barrier()

        def chunk_body(g, _):
            pltpu.sync_copy(src_hbm.at[s, pl.ds(g * CH, CH)], src_v)
            pltpu.sync_copy(dst_hbm.at[s, pl.ds(g * CH, CH)], dst_v)

            pltpu.async_copy(feat_sp.at[src_v.at[0]], rows2.at[0], gsem)

            def edge_body(j, _):
                b = lax.rem(j, 2)
                nb = 1 - b
                pltpu.make_async_copy(
                    feat_sp.at[src_v.at[j]], rows2.at[b], gsem).wait()

                @pl.when(j >= 1)
                def _():
                    pltpu.make_async_copy(
                        rows2.at[nb], acc_sp.at[dst_v.at[j - 1]], ssem).wait()

                @pl.when(j + 1 < CH)
                def _():
                    pltpu.async_copy(
                        feat_sp.at[src_v.at[j + 1]], rows2.at[nb], gsem)

                pltpu.async_copy(
                    rows2.at[b], acc_sp.at[dst_v.at[j]], ssem, add=True)

                of = jnp.ones((16,), jnp.float32)
                for t in range(EB // 16):
                    idx = dst_v[j, pl.ds(16 * t, 16)]
                    plsc.addupdate_scatter(deg_l, [idx], of)
                return 0

            lax.fori_loop(0, CH, edge_body, 0)
            lb = (CH - 1) % 2
            pltpu.make_async_copy(
                rows2.at[lb], acc_sp.at[dst_v.at[CH - 1]], ssem).wait()
            return 0

        lax.fori_loop(0, NCH, chunk_body, 0)

        def repack_body(q, _):
            deg_l2[q] = deg_l[pl.ds(16 * q, 16)]
            return 0

        lax.fori_loop(0, DROWS, repack_body, 0)

        def merge_body(kk, _):
            pltpu.sync_copy(deg_l2.at[pl.ds(128 * kk, 128)],
                            deg_sp.at[idx128.at[kk]], add=True)
            return 0

        lax.fori_loop(0, IROWS, merge_body, 0)

        plsc.subcore_barrier()

        pltpu.sync_copy(acc_sp.at[pl.ds(r0, ROWS_PT)],
                        agg_hbm.at[c, pl.ds(r0, ROWS_PT)])

        @pl.when(c == 0)
        def _():
            pltpu.sync_copy(deg_sp.at[pl.ds(s * (DROWS // NS), DROWS // NS)],
                            deg_hbm.at[pl.ds(s * (DROWS // NS), DROWS // NS)])

    return k(feat_s, src3, dst3)


def _tc_body(feat_ref, agg_ref, deg_ref, w1_ref, w2_ref, b_ref, o_ref):
    dinv = 1.0 / jnp.maximum(deg_ref[...], 1.0)
    o_ref[...] = (
        jnp.dot(feat_ref[...], w1_ref[...], preferred_element_type=jnp.float32)
        + jnp.dot(agg_ref[...] * dinv, w2_ref[...],
                  preferred_element_type=jnp.float32)
        + b_ref[...]
    )


def _tc_linear(feat, agg, deg, w1t, w2t, b2d):
    rb = 1000
    grid = (N_NODES // rb,)
    return pl.pallas_call(
        _tc_body,
        grid=grid,
        in_specs=[
            pl.BlockSpec((rb, D_IN), lambda i: (i, 0)),
            pl.BlockSpec((rb, D_IN), lambda i: (i, 0)),
            pl.BlockSpec((rb, 1), lambda i: (i, 0)),
            pl.BlockSpec((D_IN, D_OUT), lambda i: (0, 0)),
            pl.BlockSpec((D_IN, D_OUT), lambda i: (0, 0)),
            pl.BlockSpec((1, D_OUT), lambda i: (0, 0)),
        ],
        out_specs=pl.BlockSpec((rb, D_OUT), lambda i: (i, 0)),
        out_shape=jax.ShapeDtypeStruct((N_NODES, D_OUT), jnp.float32),
    )(feat, agg, deg, w1t, w2t, b2d)


def kernel(feat, edge_index, W, b):
    src3 = edge_index[0].astype(jnp.int32).reshape(NS, EROWS_PT, EB)
    dst3 = edge_index[1].astype(jnp.int32).reshape(NS, EROWS_PT, EB)
    feat_s = jnp.pad(jnp.stack([feat[:, :DH], feat[:, DH:]]),
                     ((0, 0), (0, N_PAD - N_NODES), (0, 0)))
    agg, deg = _sc_aggregate(feat_s, src3, dst3)
    agg_full = jnp.concatenate([agg[0, :N_NODES], agg[1, :N_NODES]], axis=1)
    deg2d = deg.reshape(N_PAD, 1)[:N_NODES]
    w1t = W[:, :D_IN].T
    w2t = W[:, D_IN:].T
    return _tc_linear(feat, agg_full, deg2d, w1t, w2t, b.reshape(1, D_OUT))


# speedup vs baseline: 11.1829x; 1.1675x over previous
import functools

import jax
import jax.numpy as jnp
from jax import lax
from jax.experimental import pallas as pl
from jax.experimental.pallas import tpu as pltpu, tpu_sc as plsc

N_NODES = 10000
N_EDGES = 320000
D_IN = 128
D_OUT = 128

DH = D_IN // 2
NS = 16
N_PAD = 10240
ROWS_PT = N_PAD // NS
EB = 80
EROWS = N_EDGES // EB
EROWS_PT = EROWS // NS
CH = 50
NCH = EROWS_PT // CH
RCH = 80
NRCH = ROWS_PT // RCH
DROWS = N_PAD // 16
FROWS_PT = N_NODES // NS
IROWS = N_PAD // 2048


def _sc_aggregate(feat, src3, dst3):
    mesh = plsc.VectorSubcoreMesh(core_axis_name="c", subcore_axis_name="s")

    @functools.partial(
        pl.kernel,
        out_type=(
            jax.ShapeDtypeStruct((N_PAD, D_IN), jnp.float32),
            jax.ShapeDtypeStruct((DROWS, 16), jnp.float32),
        ),
        mesh=mesh,
        scratch_types=[
            pltpu.VMEM_SHARED((N_PAD, DH), jnp.float32),
            pltpu.VMEM_SHARED((N_PAD, DH), jnp.float32),
            pltpu.VMEM_SHARED((DROWS, 16), jnp.float32),
            pltpu.VMEM((CH, EB), jnp.int32),
            pltpu.VMEM((CH, EB), jnp.int32),
            pltpu.VMEM((2, EB, DH), jnp.float32),
            pltpu.VMEM((RCH, DH), jnp.float32),
            pltpu.VMEM((N_PAD,), jnp.float32),
            pltpu.VMEM((DROWS, 16), jnp.float32),
            pltpu.VMEM((IROWS, 128), jnp.int32),
            pltpu.SemaphoreType.DMA,
            pltpu.SemaphoreType.DMA,
        ],
        compiler_params=pltpu.CompilerParams(
            use_tc_tiling_on_sc=False, needs_layout_passes=False),
    )
    def k(feat_hbm, src_hbm, dst_hbm, agg_hbm, deg_hbm,
          feat_sp, acc_sp, deg_sp, src_v, dst_v, rows2, zero_v, deg_l,
          deg_l2, idx128, gsem, ssem):
        c = lax.axis_index("c")
        s = lax.axis_index("s")
        r0 = s * ROWS_PT

        f0 = s * FROWS_PT
        pltpu.sync_copy(feat_hbm.at[pl.ds(f0, FROWS_PT), pl.ds(c * DH, DH)],
                        feat_sp.at[pl.ds(f0, FROWS_PT)])

        zf = jnp.zeros((16,), jnp.float32)

        def zero_stage(i, _):
            for j in range(DH // 16):
                zero_v[i, pl.ds(16 * j, 16)] = zf
            return 0

        lax.fori_loop(0, RCH, zero_stage, 0)

        def zero_hist(i, _):
            deg_l[pl.ds(16 * i, 16)] = zf
            deg_l2[i] = zf
            return 0

        lax.fori_loop(0, DROWS, zero_hist, 0)

        iot = lax.iota(jnp.int32, 16)

        def fill_iota(q, _):
            for t in range(8):
                idx128[q, pl.ds(16 * t, 16)] = iot + q * 128 + 16 * t
            return 0

        lax.fori_loop(0, IROWS, fill_iota, 0)

        def zero_copy(kk, _):
            pltpu.sync_copy(zero_v, acc_sp.at[pl.ds(r0 + kk * RCH, RCH)])
            return 0

        lax.fori_loop(0, NRCH, zero_copy, 0)
        pltpu.sync_copy(deg_l2.at[pl.ds(s * (DROWS // NS), DROWS // NS)],
                        deg_sp.at[pl.ds(s * (DROWS // NS), DROWS // NS)])

        plsc.subcore_barrier()

        def chunk_body(g, _):
            pltpu.sync_copy(src_hbm.at[s, pl.ds(g * CH, CH)], src_v)
            pltpu.sync_copy(dst_hbm.at[s, pl.ds(g * CH, CH)], dst_v)

            pltpu.async_copy(feat_sp.at[src_v.at[0]], rows2.at[0], gsem)

            def edge_body(j, _):
                b = lax.rem(j, 2)
                nb = 1 - b
                pltpu.make_async_copy(
                    feat_sp.at[src_v.at[j]], rows2.at[b], gsem).wait()

                @pl.when(j >= 1)
                def _():
                    pltpu.make_async_copy(
                        rows2.at[nb], acc_sp.at[dst_v.at[j - 1]], ssem).wait()

                @pl.when(j + 1 < CH)
                def _():
                    pltpu.async_copy(
                        feat_sp.at[src_v.at[j + 1]], rows2.at[nb], gsem)

                pltpu.async_copy(
                    rows2.at[b], acc_sp.at[dst_v.at[j]], ssem, add=True)

                of = jnp.ones((16,), jnp.float32)
                for t in range(EB // 16):
                    idx = dst_v[j, pl.ds(16 * t, 16)]
                    plsc.addupdate_scatter(deg_l, [idx], of)
                return 0

            lax.fori_loop(0, CH, edge_body, 0)
            lb = (CH - 1) % 2
            pltpu.make_async_copy(
                rows2.at[lb], acc_sp.at[dst_v.at[CH - 1]], ssem).wait()
            return 0

        lax.fori_loop(0, NCH, chunk_body, 0)

        def repack_body(q, _):
            deg_l2[q] = deg_l[pl.ds(16 * q, 16)]
            return 0

        lax.fori_loop(0, DROWS, repack_body, 0)

        def merge_body(kk, _):
            pltpu.sync_copy(deg_l2.at[pl.ds(128 * kk, 128)],
                            deg_sp.at[idx128.at[kk]], add=True)
            return 0

        lax.fori_loop(0, IROWS, merge_body, 0)

        plsc.subcore_barrier()

        pltpu.sync_copy(acc_sp.at[pl.ds(r0, ROWS_PT)],
                        agg_hbm.at[pl.ds(r0, ROWS_PT), pl.ds(c * DH, DH)])

        @pl.when(c == 0)
        def _():
            pltpu.sync_copy(deg_sp.at[pl.ds(s * (DROWS // NS), DROWS // NS)],
                            deg_hbm.at[pl.ds(s * (DROWS // NS), DROWS // NS)])

    return k(feat, src3, dst3)


def _tc_body(feat_ref, agg_ref, deg_ref, w1_ref, w2_ref, b_ref, o_ref):
    dinv = 1.0 / jnp.maximum(deg_ref[...], 1.0)
    o_ref[...] = (
        jnp.dot(feat_ref[...], w1_ref[...], preferred_element_type=jnp.float32)
        + jnp.dot(agg_ref[...] * dinv, w2_ref[...],
                  preferred_element_type=jnp.float32)
        + b_ref[...]
    )


def _tc_linear(feat, agg, deg, w1t, w2t, b2d):
    rb = 1000
    grid = (N_NODES // rb,)
    return pl.pallas_call(
        _tc_body,
        grid=grid,
        in_specs=[
            pl.BlockSpec((rb, D_IN), lambda i: (i, 0)),
            pl.BlockSpec((rb, D_IN), lambda i: (i, 0)),
            pl.BlockSpec((rb, 1), lambda i: (i, 0)),
            pl.BlockSpec((D_IN, D_OUT), lambda i: (0, 0)),
            pl.BlockSpec((D_IN, D_OUT), lambda i: (0, 0)),
            pl.BlockSpec((1, D_OUT), lambda i: (0, 0)),
        ],
        out_specs=pl.BlockSpec((rb, D_OUT), lambda i: (i, 0)),
        out_shape=jax.ShapeDtypeStruct((N_NODES, D_OUT), jnp.float32),
    )(feat, agg, deg, w1t, w2t, b2d)


def kernel(feat, edge_index, W, b):
    src3 = edge_index[0].astype(jnp.int32).reshape(NS, EROWS_PT, EB)
    dst3 = edge_index[1].astype(jnp.int32).reshape(NS, EROWS_PT, EB)
    agg, deg = _sc_aggregate(feat, src3, dst3)
    deg2d = deg.reshape(N_PAD, 1)[:N_NODES]
    w1t = W[:, :D_IN].T
    w2t = W[:, D_IN:].T
    return _tc_linear(feat, agg, deg2d, w1t, w2t, b.reshape(1, D_OUT))


# speedup vs baseline: 12.9366x; 1.1568x over previous
import functools

import jax
import jax.numpy as jnp
from jax import lax
from jax.experimental import pallas as pl
from jax.experimental.pallas import tpu as pltpu, tpu_sc as plsc

N_NODES = 10000
N_EDGES = 320000
D_IN = 128
D_OUT = 128

DH = D_IN // 2
NS = 16
N_PAD = 10240
ROWS_PT = N_PAD // NS
EB = 80
EROWS = N_EDGES // EB
EROWS_PT = EROWS // NS
CH = 50
NCH = EROWS_PT // CH
RCH = 80
NRCH = ROWS_PT // RCH
DROWS = N_PAD // 16
FROWS_PT = N_NODES // NS
IROWS = N_PAD // 2048


def _sc_aggregate(feat, src3, dst3):
    mesh = plsc.VectorSubcoreMesh(core_axis_name="c", subcore_axis_name="s")

    @functools.partial(
        pl.kernel,
        out_type=(
            jax.ShapeDtypeStruct((N_PAD, D_IN), jnp.float32),
            jax.ShapeDtypeStruct((DROWS, 16), jnp.float32),
        ),
        mesh=mesh,
        scratch_types=[
            pltpu.VMEM_SHARED((N_PAD, DH), jnp.float32),
            pltpu.VMEM_SHARED((N_PAD, DH), jnp.float32),
            pltpu.VMEM_SHARED((DROWS, 16), jnp.float32),
            pltpu.VMEM((CH, EB), jnp.int32),
            pltpu.VMEM((CH, EB), jnp.int32),
            pltpu.VMEM((4, EB, DH), jnp.float32),
            pltpu.VMEM((RCH, DH), jnp.float32),
            pltpu.VMEM((N_PAD,), jnp.float32),
            pltpu.VMEM((128, 16), jnp.float32),
            pltpu.VMEM((IROWS, 128), jnp.int32),
            pltpu.SemaphoreType.DMA,
            pltpu.SemaphoreType.DMA,
            pltpu.SemaphoreType.DMA,
            pltpu.SemaphoreType.DMA,
        ],
        compiler_params=pltpu.CompilerParams(
            use_tc_tiling_on_sc=False, needs_layout_passes=False),
    )
    def k(feat_hbm, src_hbm, dst_hbm, agg_hbm, deg_hbm,
          feat_sp, acc_sp, deg_sp, src_v, dst_v, rows4, zero_v, deg_l,
          deg_l2, idx128, gsem0, gsem1, ssem0, ssem1):
        c = lax.axis_index("c")
        s = lax.axis_index("s")
        r0 = s * ROWS_PT
        gsems = (gsem0, gsem1)
        ssems = (ssem0, ssem1)

        f0 = s * FROWS_PT
        pltpu.sync_copy(feat_hbm.at[pl.ds(f0, FROWS_PT), pl.ds(c * DH, DH)],
                        feat_sp.at[pl.ds(f0, FROWS_PT)])

        zf = jnp.zeros((16,), jnp.float32)

        def zero_stage(i, _):
            for j in range(DH // 16):
                zero_v[i, pl.ds(16 * j, 16)] = zf
            return 0

        lax.fori_loop(0, RCH, zero_stage, 0)

        def zero_hist(i, _):
            deg_l[pl.ds(16 * i, 16)] = zf
            return 0

        lax.fori_loop(0, DROWS, zero_hist, 0)

        def zero_hist2(i, _):
            deg_l2[i] = zf
            return 0

        lax.fori_loop(0, 128, zero_hist2, 0)

        iot = lax.iota(jnp.int32, 16)

        def fill_iota(q, _):
            for t in range(8):
                idx128[q, pl.ds(16 * t, 16)] = iot + q * 128 + 16 * t
            return 0

        lax.fori_loop(0, IROWS, fill_iota, 0)

        def zero_copy(kk, _):
            pltpu.sync_copy(zero_v, acc_sp.at[pl.ds(r0 + kk * RCH, RCH)])
            return 0

        lax.fori_loop(0, NRCH, zero_copy, 0)
        pltpu.sync_copy(deg_l2.at[pl.ds(0, DROWS // NS)],
                        deg_sp.at[pl.ds(s * (DROWS // NS), DROWS // NS)])

        plsc.subcore_barrier()

        of = jnp.ones((16,), jnp.float32)

        def chunk_body(g, _):
            pltpu.sync_copy(src_hbm.at[s, pl.ds(g * CH, CH)], src_v)
            pltpu.sync_copy(dst_hbm.at[s, pl.ds(g * CH, CH)], dst_v)

            pltpu.async_copy(feat_sp.at[src_v.at[0]], rows4.at[0], gsems[0])
            pltpu.async_copy(feat_sp.at[src_v.at[1]], rows4.at[1], gsems[1])

            def pair_body(jj, _):
                for p in range(2):
                    j = 2 * jj + p
                    gs = gsems[p]
                    ss = ssems[p]
                    b = lax.rem(j, 4)
                    bn = lax.rem(j + 2, 4)
                    pltpu.make_async_copy(
                        feat_sp.at[src_v.at[j]], rows4.at[b], gs).wait()

                    @pl.when(j >= 2)
                    def _():
                        pltpu.make_async_copy(
                            rows4.at[bn], acc_sp.at[dst_v.at[j - 2]],
                            ss).wait()

                    @pl.when(j + 2 < CH)
                    def _():
                        pltpu.async_copy(
                            feat_sp.at[src_v.at[j + 2]], rows4.at[bn], gs)

                    pltpu.async_copy(
                        rows4.at[b], acc_sp.at[dst_v.at[j]], ss, add=True)

                    for t in range(EB // 16):
                        idx = dst_v[j, pl.ds(16 * t, 16)]
                        plsc.addupdate_scatter(deg_l, [idx], of)
                return 0

            lax.fori_loop(0, CH // 2, pair_body, 0)
            for dd in (CH - 2, CH - 1):
                pltpu.make_async_copy(
                    rows4.at[dd % 4], acc_sp.at[dst_v.at[dd]],
                    ssems[dd % 2]).wait()
            return 0

        lax.fori_loop(0, NCH, chunk_body, 0)

        def merge_body(kk, _):
            def repack(q, _):
                deg_l2[q] = deg_l[pl.ds(2048 * kk + 16 * q, 16)]
                return 0

            lax.fori_loop(0, 128, repack, 0)
            pltpu.sync_copy(deg_l2, deg_sp.at[idx128.at[kk]], add=True)
            return 0

        lax.fori_loop(0, IROWS, merge_body, 0)

        plsc.subcore_barrier()

        pltpu.sync_copy(acc_sp.at[pl.ds(r0, ROWS_PT)],
                        agg_hbm.at[pl.ds(r0, ROWS_PT), pl.ds(c * DH, DH)])

        @pl.when(c == 0)
        def _():
            pltpu.sync_copy(deg_sp.at[pl.ds(s * (DROWS // NS), DROWS // NS)],
                            deg_hbm.at[pl.ds(s * (DROWS // NS), DROWS // NS)])

    return k(feat, src3, dst3)


def _tc_body(feat_ref, agg_ref, deg_ref, w1_ref, w2_ref, b_ref, o_ref):
    dinv = 1.0 / jnp.maximum(deg_ref[...], 1.0)
    o_ref[...] = (
        jnp.dot(feat_ref[...], w1_ref[...], preferred_element_type=jnp.float32)
        + jnp.dot(agg_ref[...] * dinv, w2_ref[...],
                  preferred_element_type=jnp.float32)
        + b_ref[...]
    )


def _tc_linear(feat, agg, deg, w1t, w2t, b2d):
    rb = 1000
    grid = (N_NODES // rb,)
    return pl.pallas_call(
        _tc_body,
        grid=grid,
        in_specs=[
            pl.BlockSpec((rb, D_IN), lambda i: (i, 0)),
            pl.BlockSpec((rb, D_IN), lambda i: (i, 0)),
            pl.BlockSpec((rb, 1), lambda i: (i, 0)),
            pl.BlockSpec((D_IN, D_OUT), lambda i: (0, 0)),
            pl.BlockSpec((D_IN, D_OUT), lambda i: (0, 0)),
            pl.BlockSpec((1, D_OUT), lambda i: (0, 0)),
        ],
        out_specs=pl.BlockSpec((rb, D_OUT), lambda i: (i, 0)),
        out_shape=jax.ShapeDtypeStruct((N_NODES, D_OUT), jnp.float32),
    )(feat, agg, deg, w1t, w2t, b2d)


def kernel(feat, edge_index, W, b):
    src3 = edge_index[0].astype(jnp.int32).reshape(NS, EROWS_PT, EB)
    dst3 = edge_index[1].astype(jnp.int32).reshape(NS, EROWS_PT, EB)
    agg, deg = _sc_aggregate(feat, src3, dst3)
    deg2d = deg.reshape(N_PAD, 1)[:N_NODES]
    w1t = W[:, :D_IN].T
    w2t = W[:, D_IN:].T
    return _tc_linear(feat, agg, deg2d, w1t, w2t, b.reshape(1, D_OUT))


# speedup vs baseline: 12.9962x; 1.0046x over previous
import functools

import jax
import jax.numpy as jnp
from jax import lax
from jax.experimental import pallas as pl
from jax.experimental.pallas import tpu as pltpu, tpu_sc as plsc

N_NODES = 10000
N_EDGES = 320000
D_IN = 128
D_OUT = 128

DH = D_IN // 2
NS = 16
N_PAD = 10240
ROWS_PT = N_PAD // NS
EB = 80
EROWS = N_EDGES // EB
EROWS_PT = EROWS // NS
CH = 50
NCH = EROWS_PT // CH
RCH = 80
NRCH = ROWS_PT // RCH
DROWS = N_PAD // 16
FROWS_PT = N_NODES // NS
IROWS = N_PAD // 2048


def _sc_aggregate(feat, src3, dst3):
    mesh = plsc.VectorSubcoreMesh(core_axis_name="c", subcore_axis_name="s")

    @functools.partial(
        pl.kernel,
        out_type=(
            jax.ShapeDtypeStruct((N_PAD, D_IN), jnp.float32),
            jax.ShapeDtypeStruct((DROWS, 16), jnp.float32),
        ),
        mesh=mesh,
        scratch_types=[
            pltpu.VMEM_SHARED((N_PAD, DH), jnp.float32),
            pltpu.VMEM_SHARED((N_PAD, DH), jnp.float32),
            pltpu.VMEM_SHARED((DROWS, 16), jnp.float32),
            pltpu.VMEM((CH, EB), jnp.int32),
            pltpu.VMEM((CH, EB), jnp.int32),
            pltpu.VMEM((4, EB, DH), jnp.float32),
            pltpu.VMEM((RCH, DH), jnp.float32),
            pltpu.VMEM((N_PAD,), jnp.float32),
            pltpu.VMEM((128, 16), jnp.float32),
            pltpu.VMEM((IROWS, 128), jnp.int32),
            pltpu.SemaphoreType.DMA,
            pltpu.SemaphoreType.DMA,
            pltpu.SemaphoreType.DMA,
            pltpu.SemaphoreType.DMA,
        ],
        compiler_params=pltpu.CompilerParams(
            use_tc_tiling_on_sc=False, needs_layout_passes=False),
    )
    def k(feat_hbm, src_hbm, dst_hbm, agg_hbm, deg_hbm,
          feat_sp, acc_sp, deg_sp, src_v, dst_v, rows4, zero_v, deg_l,
          deg_l2, idx128, gsem0, gsem1, ssem0, ssem1):
        c = lax.axis_index("c")
        s = lax.axis_index("s")
        r0 = s * ROWS_PT
        gsems = (gsem0, gsem1)
        ssems = (ssem0, ssem1)

        f0 = s * FROWS_PT
        pltpu.sync_copy(feat_hbm.at[pl.ds(f0, FROWS_PT), pl.ds(c * DH, DH)],
                        feat_sp.at[pl.ds(f0, FROWS_PT)])

        zf = jnp.zeros((16,), jnp.float32)

        def zero_stage(i, _):
            for j in range(DH // 16):
                zero_v[i, pl.ds(16 * j, 16)] = zf
            return 0

        lax.fori_loop(0, RCH, zero_stage, 0)

        def zero_hist(i, _):
            deg_l[pl.ds(16 * i, 16)] = zf
            return 0

        lax.fori_loop(0, DROWS, zero_hist, 0)

        def zero_hist2(i, _):
            deg_l2[i] = zf
            return 0

        lax.fori_loop(0, 128, zero_hist2, 0)

        iot = lax.iota(jnp.int32, 16)

        def fill_iota(q, _):
            for t in range(8):
                idx128[q, pl.ds(16 * t, 16)] = iot + q * 128 + 16 * t
            return 0

        lax.fori_loop(0, IROWS, fill_iota, 0)

        def zero_copy(kk, _):
            pltpu.sync_copy(zero_v, acc_sp.at[pl.ds(r0 + kk * RCH, RCH)])
            return 0

        lax.fori_loop(0, NRCH, zero_copy, 0)
        pltpu.sync_copy(deg_l2.at[pl.ds(0, DROWS // NS)],
                        deg_sp.at[pl.ds(s * (DROWS // NS), DROWS // NS)])

        plsc.subcore_barrier()

        of = jnp.ones((16,), jnp.float32)

        def chunk_body(g, _):
            pltpu.sync_copy(src_hbm.at[s, pl.ds(g * CH, CH)], src_v)
            pltpu.sync_copy(dst_hbm.at[s, pl.ds(g * CH, CH)], dst_v)

            pltpu.async_copy(feat_sp.at[src_v.at[0]], rows4.at[0], gsems[0])
            pltpu.async_copy(feat_sp.at[src_v.at[1]], rows4.at[1], gsems[1])

            def pair_body(jj, _):
                for p in range(2):
                    j = 2 * jj + p
                    gs = gsems[p]
                    ss = ssems[p]
                    b = lax.rem(j, 4)
                    bn = lax.rem(j + 2, 4)
                    pltpu.make_async_copy(
                        feat_sp.at[src_v.at[j]], rows4.at[b], gs).wait()

                    @pl.when(j >= 2)
                    def _():
                        pltpu.make_async_copy(
                            rows4.at[bn], acc_sp.at[dst_v.at[j - 2]],
                            ss).wait()

                    @pl.when(j + 2 < CH)
                    def _():
                        pltpu.async_copy(
                            feat_sp.at[src_v.at[j + 2]], rows4.at[bn], gs)

                    pltpu.async_copy(
                        rows4.at[b], acc_sp.at[dst_v.at[j]], ss, add=True)

                    for t in range(EB // 16):
                        idx = dst_v[j, pl.ds(16 * t, 16)]
                        plsc.addupdate_scatter(deg_l, [idx], of)
                return 0

            lax.fori_loop(0, CH // 2, pair_body, 0)
            for dd in (CH - 2, CH - 1):
                pltpu.make_async_copy(
                    rows4.at[dd % 4], acc_sp.at[dst_v.at[dd]],
                    ssems[dd % 2]).wait()
            return 0

        lax.fori_loop(0, NCH, chunk_body, 0)

        def merge_body(kk, _):
            def repack(q, _):
                deg_l2[q] = deg_l[pl.ds(2048 * kk + 16 * q, 16)]
                return 0

            lax.fori_loop(0, 128, repack, 0)
            pltpu.sync_copy(deg_l2, deg_sp.at[idx128.at[kk]], add=True)
            return 0

        lax.fori_loop(0, IROWS, merge_body, 0)

        plsc.subcore_barrier()

        pltpu.sync_copy(acc_sp.at[pl.ds(r0, ROWS_PT)],
                        agg_hbm.at[pl.ds(r0, ROWS_PT), pl.ds(c * DH, DH)])

        @pl.when(c == 0)
        def _():
            pltpu.sync_copy(deg_sp.at[pl.ds(s * (DROWS // NS), DROWS // NS)],
                            deg_hbm.at[pl.ds(s * (DROWS // NS), DROWS // NS)])

    return k(feat, src3, dst3)


def _tc1_body(feat_ref, w1_ref, b_ref, o_ref):
    o_ref[...] = jnp.dot(feat_ref[...], w1_ref[...],
                         preferred_element_type=jnp.float32) + b_ref[...]


def _tc2_body(p1_ref, agg_ref, deg_ref, w2_ref, o_ref):
    dinv = 1.0 / jnp.maximum(deg_ref[...], 1.0)
    o_ref[...] = p1_ref[...] + jnp.dot(
        agg_ref[...] * dinv, w2_ref[...], preferred_element_type=jnp.float32)


RB = 1000


def _tc_linear1(feat, w1t, b2d):
    grid = (N_NODES // RB,)
    return pl.pallas_call(
        _tc1_body,
        grid=grid,
        in_specs=[
            pl.BlockSpec((RB, D_IN), lambda i: (i, 0)),
            pl.BlockSpec((D_IN, D_OUT), lambda i: (0, 0)),
            pl.BlockSpec((1, D_OUT), lambda i: (0, 0)),
        ],
        out_specs=pl.BlockSpec((RB, D_OUT), lambda i: (i, 0)),
        out_shape=jax.ShapeDtypeStruct((N_NODES, D_OUT), jnp.float32),
    )(feat, w1t, b2d)


def _tc_linear2(p1, agg, deg, w2t):
    grid = (N_NODES // RB,)
    return pl.pallas_call(
        _tc2_body,
        grid=grid,
        in_specs=[
            pl.BlockSpec((RB, D_OUT), lambda i: (i, 0)),
            pl.BlockSpec((RB, D_IN), lambda i: (i, 0)),
            pl.BlockSpec((RB, 1), lambda i: (i, 0)),
            pl.BlockSpec((D_IN, D_OUT), lambda i: (0, 0)),
        ],
        out_specs=pl.BlockSpec((RB, D_OUT), lambda i: (i, 0)),
        out_shape=jax.ShapeDtypeStruct((N_NODES, D_OUT), jnp.float32),
    )(p1, agg, deg, w2t)


def kernel(feat, edge_index, W, b):
    src3 = edge_index[0].astype(jnp.int32).reshape(NS, EROWS_PT, EB)
    dst3 = edge_index[1].astype(jnp.int32).reshape(NS, EROWS_PT, EB)
    agg, deg = _sc_aggregate(feat, src3, dst3)
    p1 = _tc_linear1(feat, W[:, :D_IN].T, b.reshape(1, D_OUT))
    deg2d = deg.reshape(N_PAD, 1)[:N_NODES]
    return _tc_linear2(p1, agg, deg2d, W[:, D_IN:].T)
